# Initial kernel scaffold; baseline (speedup 1.0000x reference)
import jax
import jax.numpy as jnp
from jax.experimental import pallas as pl


def kernel(x, edge_index, edge_attr, batch, gamma, beta, Wr1, br1, Wo1, attn1, Wr2, br2, Wo2, attn2, Wr3, br3, Wo3, attn3, Wh1, bh1, Wh2, bh2, Wh3, bh3):
    raise NotImplementedError("write your pallas kernel here")


# speedup vs baseline: 26.4911x; 26.4911x over previous
import functools

import jax
import jax.numpy as jnp
from jax import lax
from jax.experimental import pallas as pl
from jax.experimental.pallas import tpu as pltpu
from jax.experimental.pallas import tpu_sc as plsc

N = 10000
E = 320000
D = 128
H = 30
HP = 32
NHEADS = 11

NC, NS, L = 2, 16, 16
NW = NC * NS
CHK = 128
NCHUNK = -(-(E // NW) // CHK)
EPAD = NW * NCHUNK * CHK
NP_ = ((N + NS - 1) // NS + 7) // 8 * 8 * NS
RPT = NP_ // NS

K1 = -(-9 * N // 10)
K2 = -(-9 * K1 // 10)
K3 = -(-9 * K2 // 10)

def _make_conv_body(dw):
    def _conv_edges_body(p_hbm, srcr, dstr, wexp, out_hbm,
                         src_v, dst_v, rows_v, wrow_v, zb_v, agg_sh, sem):
        c = lax.axis_index("c")
        s = lax.axis_index("s")
        wid = s * NC + c
        zero16 = jnp.zeros((L,), jnp.float32)

        def zb_body(i, carry):
            for q in range(dw // L):
                zb_v[i, pl.ds(q * L, L)] = zero16
            return carry

        lax.fori_loop(0, CHK, zb_body, 0)
        for off in range(0, RPT, CHK):
            n = min(CHK, RPT - off)
            pltpu.sync_copy(zb_v.at[pl.ds(0, n)],
                            agg_sh.at[pl.ds(s * RPT + off, n)])

        pltpu.sync_copy(srcr.at[wid], src_v)
        pltpu.sync_copy(dstr.at[wid], dst_v)
        plsc.subcore_barrier()

        base_e = wid * (NCHUNK * CHK)

        def chunk_body(j, carry):
            pltpu.async_copy(p_hbm.at[src_v.at[j]], rows_v, sem).wait()
            pltpu.sync_copy(wexp.at[pl.ds(base_e + j * CHK, CHK)], wrow_v)

            def scale_body(e, cc):
                w16 = wrow_v[e, pl.ds(0, L)]
                for q in range(dw // L):
                    rows_v[e, pl.ds(q * L, L)] = (rows_v[e, pl.ds(q * L, L)]
                                                  * w16)
                return cc

            lax.fori_loop(0, CHK, scale_body, 0)
            pltpu.sync_copy(rows_v, agg_sh.at[dst_v.at[j]], add=True)
            return carry

        lax.fori_loop(0, NCHUNK, chunk_body, 0)
        plsc.subcore_barrier()
        pltpu.sync_copy(agg_sh.at[pl.ds(s * RPT, RPT)],
                        out_hbm.at[pl.ds(c * NP_ + s * RPT, RPT)])

    return _conv_edges_body


@functools.cache
def _conv_edges(dw):
    mesh = plsc.VectorSubcoreMesh(core_axis_name="c", subcore_axis_name="s",
                                  num_cores=NC, num_subcores=NS)
    return pl.kernel(
        _make_conv_body(dw),
        out_type=jax.ShapeDtypeStruct((NC * NP_, dw), jnp.float32),
        mesh=mesh,
        compiler_params=pltpu.CompilerParams(use_tc_tiling_on_sc=False),
        scratch_types=[
            pltpu.VMEM((NCHUNK, CHK), jnp.int32),
            pltpu.VMEM((NCHUNK, CHK), jnp.int32),
            pltpu.VMEM((CHK, dw), jnp.float32),
            pltpu.VMEM((CHK, HP), jnp.float32),
            pltpu.VMEM((CHK, dw), jnp.float32),
            pltpu.VMEM_SHARED((NP_, dw), jnp.float32),
            pltpu.SemaphoreType.DMA,
        ],
    )


NP4 = NP_ // 4
DP = 4 * D


def _dot(a, b):
    return jnp.dot(a, b, preferred_element_type=jnp.float32,
                   precision=lax.Precision.DEFAULT)


def _doth(a, b):
    return jnp.dot(a, b, preferred_element_type=jnp.float32,
                   precision=lax.Precision.HIGHEST)


def _pre_kernel(x_ref, mdiv_ref, g_ref, b_ref, wr_ref, br_ref, wo_ref,
                p_ref, r_ref):
    xx = x_ref[...]
    mu = _doth(xx, mdiv_ref[...])
    xc = xx - mu
    var = _doth(xc * xc, mdiv_ref[...])
    h = xc / jnp.sqrt(var + 1e-5) * g_ref[...] + b_ref[...]
    p_ref[...] = _dot(h, wr_ref[...])
    r_ref[...] = _dot(h, wo_ref[...]) + br_ref[...]


def _mid(lo, hi):
    return (lo >> 1) + (hi >> 1) + (lo & hi & 1)


def _select_and_pool(u, m_ref, ablk_ref, g14_ref, k):
    a = ablk_ref[...]
    norm = jnp.sqrt(jnp.sum(a * a) * 0.25)
    s4 = _dot(u, ablk_ref[...]) / (norm + 1e-16)
    t4 = jnp.tanh(s4)
    m4 = m_ref[...]
    sm = jnp.where(m4 > 0.5, s4, jnp.float32(-3e38))
    bits = lax.bitcast_convert_type(sm, jnp.int32)
    key = jnp.where(bits >= 0, bits, bits ^ jnp.int32(0x7FFFFFFF))

    def bs_val(_, lohi):
        lo, hi = lohi
        mid = _mid(lo, hi)
        cnt = jnp.sum((key >= mid).astype(jnp.int32))
        return jnp.where(cnt >= k, mid, lo), jnp.where(cnt >= k, hi, mid)

    tau, _ = lax.fori_loop(0, 32, bs_val,
                           (jnp.int32(-2**31), jnp.int32(2**31 - 1)))
    c_gt = jnp.sum((key > tau).astype(jnp.int32))
    tie = key == tau
    idx = (lax.broadcasted_iota(jnp.int32, (NP4, 4), 0) * 4
           + lax.broadcasted_iota(jnp.int32, (NP4, 4), 1))

    def bs_idx(_, lohi):
        lo, hi = lohi
        mid = lo + (hi - lo) // 2
        cnt = c_gt + jnp.sum((tie & (idx < mid)).astype(jnp.int32))
        return jnp.where(cnt >= k, lo, mid), jnp.where(cnt >= k, mid, hi)

    _, cut = lax.fori_loop(0, 15, bs_idx, (jnp.int32(-1), jnp.int32(16384)))
    mn4 = ((key > tau) | (tie & (idx < cut))).astype(jnp.float32)
    tl = _doth(t4, g14_ref[...])
    mnl = _doth(mn4, g14_ref[...])
    hn = u * tl * mnl
    rmaxl = jnp.max(jnp.where(mnl > 0.5, hn, jnp.float32(-3e38)),
                    axis=0, keepdims=True)
    rsuml = jnp.sum(hn, axis=0, keepdims=True)
    rmax = rmaxl[:, 0:HP]
    rsum = rsuml[:, 0:HP]
    for g in range(1, 4):
        rmax = jnp.maximum(rmax, rmaxl[:, g * HP:(g + 1) * HP])
        rsum = rsum + rsuml[:, g * HP:(g + 1) * HP]
    rsum = rsum * jnp.float32(1.0 / k)
    return hn, mn4, rmax, rsum


def _post1_kernel(k, agg_ref, r_ref, m_ref, ablk_ref, g14_ref,
                  hn_ref, mn_ref, read_ref):
    agg = agg_ref[0:NP4, :] + agg_ref[NP4:2 * NP4, :]
    u = jnp.maximum(agg + r_ref[...], 0.0)
    hn, mn4, rmax, rsum = _select_and_pool(u, m_ref, ablk_ref, g14_ref, k)
    hn_ref[...] = hn
    mn_ref[...] = mn4
    read_ref[0:1, :] = rmax
    read_ref[1:2, :] = rsum


def _conv_u(agg_ref, h_ref, wr_ref, br_ref, wo_ref):
    agg = agg_ref[0:NP4, :] + agg_ref[NP4:2 * NP4, :]
    return jnp.maximum(_dot(agg, wr_ref[...]) + br_ref[...]
                       + _dot(h_ref[...], wo_ref[...]), 0.0)


def _post_kernel(k, agg_ref, h_ref, m_ref, wrn_ref, brn_ref, won_ref,
                 ablk_ref, g14_ref, hn_ref, mn_ref, read_ref):
    u = _conv_u(agg_ref, h_ref, wrn_ref, brn_ref, won_ref)
    hn, mn4, rmax, rsum = _select_and_pool(u, m_ref, ablk_ref, g14_ref, k)
    hn_ref[...] = hn
    mn_ref[...] = mn4
    read_ref[0:1, :] = rmax
    read_ref[1:2, :] = rsum


def _final_kernel(k, agg_ref, h_ref, m_ref, wrn_ref, brn_ref, won_ref,
                  ablk_ref, g14_ref, rd1_ref, rd2_ref,
                  wh1_ref, bh1_ref, wh2_ref, bh2_ref, wh3_ref, bh3_ref,
                  out_ref):
    u = _conv_u(agg_ref, h_ref, wrn_ref, brn_ref, won_ref)
    _, _, rmax, rsum = _select_and_pool(u, m_ref, ablk_ref, g14_ref, k)
    x3 = jnp.concatenate([rmax[:, 0:H], rsum[:, 0:H]], axis=1)
    x1 = jnp.concatenate([rd1_ref[0:1, 0:H], rd1_ref[1:2, 0:H]], axis=1)
    x2 = jnp.concatenate([rd2_ref[0:1, 0:H], rd2_ref[1:2, 0:H]], axis=1)
    z0 = jnp.maximum(x1 + x2 + x3, 0.0)
    for i in range(NHEADS):
        z1 = jnp.maximum(
            jnp.dot(z0, wh1_ref[i], preferred_element_type=jnp.float32, precision=lax.Precision.DEFAULT)
            + bh1_ref[i:i + 1, :], 0.0)
        z2 = jnp.maximum(
            jnp.dot(z1, wh2_ref[i], preferred_element_type=jnp.float32, precision=lax.Precision.DEFAULT)
            + bh2_ref[i:i + 1, :], 0.0)
        z3 = (jnp.dot(z2, wh3_ref[i], preferred_element_type=jnp.float32, precision=lax.Precision.DEFAULT)
              + bh3_ref[i:i + 1, :])
        out_ref[i:i + 1, :] = z3


_TC_PARAMS = pltpu.CompilerParams(vmem_limit_bytes=100 * 1024 * 1024)


def _pad_w(w):
    r, c = w.shape
    return jnp.pad(w, ((0, HP - r if r == H else 0), (0, HP - c)))


def kernel(x, edge_index, edge_attr, batch, gamma, beta,
           Wr1, br1, Wo1, attn1, Wr2, br2, Wo2, attn2, Wr3, br3, Wo3, attn3,
           Wh1, bh1, Wh2, bh2, Wh3, bh3):
    f32 = jnp.float32
    eye4 = jnp.eye(4, dtype=f32)
    xq = jnp.pad(x, ((0, NP_ - N), (0, 0))).reshape(NP4, DP)
    mdiv = jnp.kron(eye4, jnp.full((D, D), 1.0 / D, f32))
    g14 = jnp.kron(eye4, jnp.ones((1, HP), f32))
    gt = jnp.tile(gamma.reshape(1, D), (1, 4))
    bt = jnp.tile(beta.reshape(1, D), (1, 4))
    wr = [jnp.kron(eye4, jnp.pad(Wr1, ((0, 0), (0, HP - H)))),
          jnp.kron(eye4, _pad_w(Wr2)), jnp.kron(eye4, _pad_w(Wr3))]
    wo = [jnp.kron(eye4, jnp.pad(Wo1, ((0, 0), (0, HP - H)))),
          jnp.kron(eye4, _pad_w(Wo2)), jnp.kron(eye4, _pad_w(Wo3))]
    br = [jnp.tile(jnp.pad(b.reshape(1, H), ((0, 0), (0, HP - H))), (1, 4))
          for b in (br1, br2, br3)]
    ablk = [jnp.kron(eye4, jnp.pad(a.reshape(H, 1), ((0, HP - H), (0, 0))))
            for a in (attn1, attn2, attn3)]

    srcr = jnp.pad(edge_index[0], (0, EPAD - E)).reshape(NW, NCHUNK, CHK)
    dstr = jnp.pad(edge_index[1], (0, EPAD - E)).reshape(NW, NCHUNK, CHK)
    wexp = jnp.broadcast_to(jnp.pad(edge_attr, (0, EPAD - E))[:, None],
                            (EPAD, HP))

    m0 = (jnp.arange(NP_, dtype=jnp.int32) < N).astype(f32).reshape(NP4, 4)

    nsd = jax.ShapeDtypeStruct
    p, r = pl.pallas_call(
        _pre_kernel,
        out_shape=[nsd((NP4, 4 * HP), f32), nsd((NP4, 4 * HP), f32)],
        compiler_params=_TC_PARAMS,
    )(xq, mdiv, gt, bt, wr[0], br[0], wo[0])

    agg = _conv_edges(HP)(p.reshape(NP_, HP), srcr, dstr, wexp)
    h, m, rd1 = pl.pallas_call(
        functools.partial(_post1_kernel, K1),
        out_shape=[nsd((NP4, 4 * HP), f32),
                   nsd((NP4, 4), f32), nsd((2, HP), f32)],
        compiler_params=_TC_PARAMS,
    )(agg.reshape(2 * NP4, 4 * HP), r, m0, ablk[0], g14)

    agg = _conv_edges(HP)(h.reshape(NP_, HP), srcr, dstr, wexp)
    h, m, rd2 = pl.pallas_call(
        functools.partial(_post_kernel, K2),
        out_shape=[nsd((NP4, 4 * HP), f32),
                   nsd((NP4, 4), f32), nsd((2, HP), f32)],
        compiler_params=_TC_PARAMS,
    )(agg.reshape(2 * NP4, 4 * HP), h, m,
      wr[1], br[1], wo[1], ablk[1], g14)

    agg = _conv_edges(HP)(h.reshape(NP_, HP), srcr, dstr, wexp)
    out = pl.pallas_call(
        functools.partial(_final_kernel, K3),
        out_shape=nsd((NHEADS, 4), f32),
        compiler_params=_TC_PARAMS,
    )(agg.reshape(2 * NP4, 4 * HP), h, m,
      wr[2], br[2], wo[2], ablk[2], g14,
      rd1, rd2, Wh1, bh1, Wh2, bh2, Wh3, bh3)
    return tuple(out[i:i + 1] for i in range(NHEADS))
